# baseline (speedup 1.0000x reference)
import functools

import jax
import jax.numpy as jnp
from jax import lax
from jax.experimental import pallas as pl
from jax.experimental.pallas import tpu as pltpu
from jax.experimental.pallas import tpu_sc as plsc

N = 10000
E = 320000
F = 128
H = 64
G = 64
HW = 128
NC = 2
NS = 16
NW = NC * NS
CHUNK = 64
NJ = 168
RING = 6
HRING = RING // 2
EPAD = NW * NJ * CHUNK
NP = 10112
RPT = NP // NS
DEGW = 16
PRS = 320
PTAIL = N - (NW - 1) * PRS

_mesh = plsc.VectorSubcoreMesh(core_axis_name="c", subcore_axis_name="s")
_sc_params = pltpu.CompilerParams(use_tc_tiling_on_sc=False)


def _deg_body(dst_hbm, ones_hbm, zeros_hbm, parts_hbm, dst_v, ones_v, *rest):
    dsem = rest[:RING]
    acc_sh = rest[RING]
    c = lax.axis_index("c")
    s = lax.axis_index("s")
    wid = s * NC + c
    pltpu.sync_copy(dst_hbm.at[wid], dst_v)
    pltpu.sync_copy(ones_hbm, ones_v)
    row0 = s * RPT
    pltpu.sync_copy(zeros_hbm, acc_sh.at[pl.ds(row0, RPT)])
    plsc.subcore_barrier()

    def start_scatter(j, k):
        pltpu.async_copy(ones_v, acc_sh.at[dst_v.at[j]], dsem[k], add=True)

    def wait_scatter(j, k):
        pltpu.make_async_copy(ones_v, acc_sh.at[dst_v.at[j]],
                              dsem[k]).wait()

    for k in range(RING):
        start_scatter(k, k)

    def body(t, carry):
        for k in range(RING):
            j = RING + t * RING + k
            wait_scatter(j - RING, k)
            start_scatter(j, k)
        return carry

    lax.fori_loop(0, (NJ - RING) // RING, body, 0)
    for j in range(NJ - RING, NJ):
        wait_scatter(j, j % RING)

    plsc.subcore_barrier()
    pltpu.sync_copy(acc_sh.at[pl.ds(row0, RPT)],
                    parts_hbm.at[pl.ds(row0, RPT), pl.ds(DEGW * c, DEGW)])


_deg_call = functools.partial(
    pl.kernel,
    mesh=_mesh,
    compiler_params=_sc_params,
    out_type=jax.ShapeDtypeStruct((NP, HW), jnp.float32),
    scratch_types=(
        [pltpu.VMEM((NJ, CHUNK), jnp.int32),
         pltpu.VMEM((CHUNK, DEGW), jnp.float32)]
        + [pltpu.SemaphoreType.DMA for _ in range(RING)]
        + [pltpu.VMEM_SHARED((NP, DEGW), jnp.float32)]
    ),
)(_deg_body)


def _agg_body(u_hbm, src_hbm, dst_hbm, zeros_hbm, parts_hbm,
              src_v, dst_v, *rest):
    bufs = rest[:RING]
    gsem = rest[RING:2 * RING]
    ssem = rest[2 * RING:3 * RING]
    u_sh = rest[3 * RING]
    acc_sh = rest[3 * RING + 1]
    c = lax.axis_index("c")
    s = lax.axis_index("s")
    wid = s * NC + c
    pltpu.sync_copy(src_hbm.at[wid], src_v)
    pltpu.sync_copy(dst_hbm.at[wid], dst_v)
    urow0 = s * (N // NS)
    pltpu.sync_copy(u_hbm.at[pl.ds(urow0, N // NS), pl.ds(0, H)],
                    u_sh.at[pl.ds(urow0, N // NS)])
    row0 = s * RPT
    pltpu.sync_copy(zeros_hbm, acc_sh.at[pl.ds(row0, RPT)])
    plsc.subcore_barrier()

    def start_gather(j, k):
        pltpu.async_copy(u_sh.at[src_v.at[j]], bufs[k], gsem[k])

    def wait_gather(j, k):
        pltpu.make_async_copy(u_sh.at[src_v.at[j]], bufs[k], gsem[k]).wait()

    def start_scatter(j, k):
        pltpu.async_copy(bufs[k], acc_sh.at[dst_v.at[j]], ssem[k], add=True)

    def wait_scatter(j, k):
        pltpu.make_async_copy(bufs[k], acc_sh.at[dst_v.at[j]],
                              ssem[k]).wait()

    for k in range(HRING):
        start_gather(k, k)
    for j in range(HRING):
        wait_gather(j, j)
        start_scatter(j, j)
        start_gather(j + HRING, j + HRING)

    def body(t, carry):
        for k in range(RING):
            j = HRING + t * RING + k
            kj = (HRING + k) % RING
            wait_gather(j, kj)
            start_scatter(j, kj)
            wait_scatter(j - HRING, k)
            start_gather(j + HRING, k)
        return carry

    lax.fori_loop(0, (NJ - RING) // RING, body, 0)

    for j in range(NJ - HRING, NJ):
        wait_gather(j, j % RING)
        start_scatter(j, j % RING)
    for j in range(NJ - RING, NJ):
        wait_scatter(j, j % RING)

    plsc.subcore_barrier()
    pltpu.sync_copy(acc_sh.at[pl.ds(row0, RPT)],
                    parts_hbm.at[pl.ds(row0, RPT), pl.ds(H * c, H)])


_agg_call = functools.partial(
    pl.kernel,
    mesh=_mesh,
    compiler_params=_sc_params,
    out_type=jax.ShapeDtypeStruct((NP, HW), jnp.float32),
    scratch_types=(
        [pltpu.VMEM((NJ, CHUNK), jnp.int32),
         pltpu.VMEM((NJ, CHUNK), jnp.int32)]
        + [pltpu.VMEM((CHUNK, H), jnp.float32) for _ in range(RING)]
        + [pltpu.SemaphoreType.DMA for _ in range(2 * RING)]
        + [pltpu.VMEM_SHARED((N, H), jnp.float32),
           pltpu.VMEM_SHARED((NP, H), jnp.float32)]
    ),
)(_agg_body)


def _pool_rows(p_v, u_v, b_v, d_v, b2_v, pooled_v, ngroups):
    def grp(t, carry):
        base = 16 * t
        bvec = b_v[pl.ds(base, 16)]
        for lane in range(16):
            i = base + lane
            g = bvec[lane]
            dinv = d_v[i, pl.ds(0, 16)][0]
            for k in range(H // 16):
                cl = 16 * k
                v = (p_v[i, pl.ds(cl, 16)] + p_v[i, pl.ds(H + cl, 16)]
                     + u_v[i, pl.ds(cl, 16)])
                h2 = jnp.maximum(v * dinv + b2_v[pl.ds(cl, 16)], 0.0)
                pooled_v[g, pl.ds(cl, 16)] = jnp.maximum(
                    pooled_v[g, pl.ds(cl, 16)], h2)
        return carry

    lax.fori_loop(0, ngroups, grp, 0)


def _pool_body(parts_hbm, u_hbm, batch_hbm, dinv_hbm, b2_hbm, out_hbm,
               p_v, u_v, b_v, d_v, b2_v, pooled_v):
    c = lax.axis_index("c")
    s = lax.axis_index("s")
    wid = s * NC + c
    row0 = wid * PRS
    pltpu.sync_copy(b2_hbm, b2_v)

    neg = jnp.full((16,), -jnp.inf, jnp.float32)
    for g in range(G):
        for k in range(H // 16):
            pooled_v[g, pl.ds(16 * k, 16)] = neg

    @pl.when(wid < NW - 1)
    def _full():
        pltpu.sync_copy(parts_hbm.at[pl.ds(row0, PRS)], p_v)
        pltpu.sync_copy(u_hbm.at[pl.ds(row0, PRS)], u_v)
        pltpu.sync_copy(batch_hbm.at[pl.ds(row0, PRS)], b_v)
        pltpu.sync_copy(dinv_hbm.at[pl.ds(row0, PRS), pl.ds(0, 16)], d_v)
        _pool_rows(p_v, u_v, b_v, d_v, b2_v, pooled_v, PRS // 16)

    @pl.when(wid == NW - 1)
    def _tail():
        pltpu.sync_copy(parts_hbm.at[pl.ds(row0, PTAIL)],
                        p_v.at[pl.ds(0, PTAIL)])
        pltpu.sync_copy(u_hbm.at[pl.ds(row0, PTAIL)],
                        u_v.at[pl.ds(0, PTAIL)])
        pltpu.sync_copy(batch_hbm.at[pl.ds(row0, PTAIL)],
                        b_v.at[pl.ds(0, PTAIL)])
        pltpu.sync_copy(dinv_hbm.at[pl.ds(row0, PTAIL), pl.ds(0, 16)],
                        d_v.at[pl.ds(0, PTAIL)])
        _pool_rows(p_v, u_v, b_v, d_v, b2_v, pooled_v, PTAIL // 16)

    pltpu.sync_copy(pooled_v, out_hbm.at[wid, pl.ds(0, G), pl.ds(0, H)])


_pool_call = functools.partial(
    pl.kernel,
    mesh=_mesh,
    compiler_params=_sc_params,
    out_type=jax.ShapeDtypeStruct((NW, G, HW), jnp.float32),
    scratch_types=[
        pltpu.VMEM((PRS, HW), jnp.float32),
        pltpu.VMEM((PRS, HW), jnp.float32),
        pltpu.VMEM((PRS,), jnp.int32),
        pltpu.VMEM((PRS, 16), jnp.float32),
        pltpu.VMEM((H,), jnp.float32),
        pltpu.VMEM((G, H), jnp.float32),
    ],
)(_pool_body)


def _mm1_body(x_ref, w1_ref, p1_ref):
    p1_ref[...] = jnp.dot(x_ref[...], w1_ref[...],
                          preferred_element_type=jnp.float32)


_mm1 = pl.pallas_call(
    _mm1_body,
    out_shape=jax.ShapeDtypeStruct((N, HW), jnp.float32),
)


def _lin1_body(p1_ref, dp_ref, u1_ref, dinv_ref):
    deg = 1.0 + dp_ref[:N, 0:1] + dp_ref[:N, DEGW:DEGW + 1]
    dinv = lax.rsqrt(deg)
    dinv_ref[...] = jnp.broadcast_to(dinv, (N, HW))
    u1_ref[...] = dinv * p1_ref[...]


_lin1 = pl.pallas_call(
    _lin1_body,
    out_shape=(jax.ShapeDtypeStruct((N, HW), jnp.float32),
               jax.ShapeDtypeStruct((N, HW), jnp.float32)),
)


def _lin2_body(p_ref, u1_ref, dinv_ref, b1_ref, w2_ref, u2_ref):
    sagg = p_ref[:N, :H] + p_ref[:N, H:]
    dinv = dinv_ref[:, 0:1]
    h = jnp.maximum(dinv * (sagg + u1_ref[:, :H]) + b1_ref[...], 0.0)
    u2_ref[...] = dinv * jnp.dot(h, w2_ref[...],
                                 preferred_element_type=jnp.float32)


_lin2 = pl.pallas_call(
    _lin2_body,
    out_shape=jax.ShapeDtypeStruct((N, HW), jnp.float32),
)


def _head_body(p_ref, w3_ref, b3_ref, out_ref):
    pooled = p_ref[0, :, :H]
    for w in range(1, NW):
        pooled = jnp.maximum(pooled, p_ref[w, :, :H])
    out_ref[...] = jnp.dot(pooled, w3_ref[...],
                           preferred_element_type=jnp.float32) + b3_ref[...]


_head = pl.pallas_call(
    _head_body,
    out_shape=jax.ShapeDtypeStruct((G, 1), jnp.float32),
)


def kernel(x, edge_index, batch_index, W1, b1, W2, b2, W3, b3):
    src = edge_index[0]
    dst = edge_index[1]
    srcp = jnp.pad(src, (0, EPAD - E)).reshape(NW, NJ, CHUNK)
    dstp = jnp.pad(dst, (0, EPAD - E),
                   constant_values=N).reshape(NW, NJ, CHUNK)
    ones = jnp.ones((CHUNK, DEGW), jnp.float32)
    zeros_deg = jnp.zeros((RPT, DEGW), jnp.float32)
    zeros_h = jnp.zeros((RPT, H), jnp.float32)
    W1p = jnp.pad(W1, ((0, 0), (0, HW - H)))
    W2p = jnp.pad(W2, ((0, 0), (0, HW - H)))

    p1 = _mm1(x, W1p)
    deg_parts = _deg_call(dstp, ones, zeros_deg)
    u1, dinv = _lin1(p1, deg_parts)
    s1 = _agg_call(u1, srcp, dstp, zeros_h)
    u2 = _lin2(s1, u1, dinv, b1.reshape(1, H), W2p)
    s2 = _agg_call(u2, srcp, dstp, zeros_h)
    pooled_parts = _pool_call(s2, u2, batch_index, dinv, b2)
    out = _head(pooled_parts, W3, b3.reshape(1, 1))
    return out


# speedup vs baseline: 1.0037x; 1.0037x over previous
import functools

import jax
import jax.numpy as jnp
from jax import lax
from jax.experimental import pallas as pl
from jax.experimental.pallas import tpu as pltpu
from jax.experimental.pallas import tpu_sc as plsc

N = 10000
E = 320000
F = 128
H = 64
G = 64
HW = 128
NC = 2
NS = 16
NW = NC * NS
CHUNK = 64
NJ = 168
RING = 6
HRING = RING // 2
EPAD = NW * NJ * CHUNK
NP = 10112
RPT = NP // NS
DEGW = 16
PRS = 320
PTAIL = N - (NW - 1) * PRS

_mesh = plsc.VectorSubcoreMesh(core_axis_name="c", subcore_axis_name="s")
_sc_params = pltpu.CompilerParams(use_tc_tiling_on_sc=False)


def _deg_body(dst_hbm, ones_hbm, zeros_hbm, parts_hbm, dst_v, ones_v, *rest):
    dsem = rest[:RING]
    acc_sh = rest[RING]
    c = lax.axis_index("c")
    s = lax.axis_index("s")
    wid = s * NC + c
    pltpu.sync_copy(dst_hbm.at[wid], dst_v)
    pltpu.sync_copy(ones_hbm, ones_v)
    row0 = s * RPT
    pltpu.sync_copy(zeros_hbm, acc_sh.at[pl.ds(row0, RPT)])
    plsc.subcore_barrier()

    def start_scatter(j, k):
        pltpu.async_copy(ones_v, acc_sh.at[dst_v.at[j]], dsem[k], add=True)

    def wait_scatter(j, k):
        pltpu.make_async_copy(ones_v, acc_sh.at[dst_v.at[j]],
                              dsem[k]).wait()

    for k in range(RING):
        start_scatter(k, k)

    def body(t, carry):
        for k in range(RING):
            j = RING + t * RING + k
            wait_scatter(j - RING, k)
            start_scatter(j, k)
        return carry

    lax.fori_loop(0, (NJ - RING) // RING, body, 0)
    for j in range(NJ - RING, NJ):
        wait_scatter(j, j % RING)

    plsc.subcore_barrier()
    pltpu.sync_copy(acc_sh.at[pl.ds(row0, RPT)],
                    parts_hbm.at[pl.ds(row0, RPT), pl.ds(DEGW * c, DEGW)])


_deg_call = functools.partial(
    pl.kernel,
    mesh=_mesh,
    compiler_params=_sc_params,
    out_type=jax.ShapeDtypeStruct((NP, HW), jnp.float32),
    scratch_types=(
        [pltpu.VMEM((NJ, CHUNK), jnp.int32),
         pltpu.VMEM((CHUNK, DEGW), jnp.float32)]
        + [pltpu.SemaphoreType.DMA for _ in range(RING)]
        + [pltpu.VMEM_SHARED((NP, DEGW), jnp.float32)]
    ),
)(_deg_body)


def _agg_body(u_hbm, src_hbm, dst_hbm, zeros_hbm, parts_hbm,
              src_v, dst_v, *rest):
    bufs = rest[:RING]
    gsem = rest[RING:2 * RING]
    ssem = rest[2 * RING:3 * RING]
    u_sh = rest[3 * RING]
    acc_sh = rest[3 * RING + 1]
    c = lax.axis_index("c")
    s = lax.axis_index("s")
    wid = s * NC + c
    pltpu.sync_copy(src_hbm.at[wid], src_v)
    pltpu.sync_copy(dst_hbm.at[wid], dst_v)
    urow0 = s * (N // NS)
    pltpu.sync_copy(u_hbm.at[pl.ds(urow0, N // NS), pl.ds(0, H)],
                    u_sh.at[pl.ds(urow0, N // NS)])
    row0 = s * RPT
    pltpu.sync_copy(zeros_hbm, acc_sh.at[pl.ds(row0, RPT)])
    plsc.subcore_barrier()

    def start_gather(j, k):
        pltpu.async_copy(u_sh.at[src_v.at[j]], bufs[k], gsem[k])

    def wait_gather(j, k):
        pltpu.make_async_copy(u_sh.at[src_v.at[j]], bufs[k], gsem[k]).wait()

    def start_scatter(j, k):
        pltpu.async_copy(bufs[k], acc_sh.at[dst_v.at[j]], ssem[k], add=True)

    def wait_scatter(j, k):
        pltpu.make_async_copy(bufs[k], acc_sh.at[dst_v.at[j]],
                              ssem[k]).wait()

    for k in range(HRING):
        start_gather(k, k)
    for j in range(HRING):
        wait_gather(j, j)
        start_scatter(j, j)
        start_gather(j + HRING, j + HRING)

    def body(t, carry):
        for k in range(RING):
            j = HRING + t * RING + k
            kj = (HRING + k) % RING
            wait_gather(j, kj)
            start_scatter(j, kj)
            wait_scatter(j - HRING, k)
            start_gather(j + HRING, k)
        return carry

    lax.fori_loop(0, (NJ - RING) // RING, body, 0)

    for j in range(NJ - HRING, NJ):
        wait_gather(j, j % RING)
        start_scatter(j, j % RING)
    for j in range(NJ - RING, NJ):
        wait_scatter(j, j % RING)

    plsc.subcore_barrier()
    pltpu.sync_copy(acc_sh.at[pl.ds(row0, RPT)],
                    parts_hbm.at[pl.ds(row0, RPT), pl.ds(H * c, H)])


_agg_call = functools.partial(
    pl.kernel,
    mesh=_mesh,
    compiler_params=_sc_params,
    out_type=jax.ShapeDtypeStruct((NP, HW), jnp.float32),
    scratch_types=(
        [pltpu.VMEM((NJ, CHUNK), jnp.int32),
         pltpu.VMEM((NJ, CHUNK), jnp.int32)]
        + [pltpu.VMEM((CHUNK, H), jnp.float32) for _ in range(RING)]
        + [pltpu.SemaphoreType.DMA for _ in range(2 * RING)]
        + [pltpu.VMEM_SHARED((N, H), jnp.float32),
           pltpu.VMEM_SHARED((NP, H), jnp.float32)]
    ),
)(_agg_body)


def _pool_rows(p_v, u_v, b_v, d_v, b2_v, pooled_v, ngroups):
    def grp(t, carry):
        base = 16 * t
        bvec = b_v[pl.ds(base, 16)]
        for lane in range(16):
            i = base + lane
            g = bvec[lane]
            dinv = d_v[i, pl.ds(0, 16)][0]
            for k in range(H // 16):
                cl = 16 * k
                v = (p_v[i, pl.ds(cl, 16)] + p_v[i, pl.ds(H + cl, 16)]
                     + u_v[i, pl.ds(cl, 16)])
                h2 = jnp.maximum(v * dinv + b2_v[pl.ds(cl, 16)], 0.0)
                pooled_v[g, pl.ds(cl, 16)] = jnp.maximum(
                    pooled_v[g, pl.ds(cl, 16)], h2)
        return carry

    lax.fori_loop(0, ngroups, grp, 0)


def _pool_body(parts_hbm, u_hbm, batch_hbm, dinv_hbm, b2_hbm, out_hbm,
               p_v, u_v, b_v, d_v, b2_v, pooled_v):
    c = lax.axis_index("c")
    s = lax.axis_index("s")
    wid = s * NC + c
    row0 = wid * PRS
    pltpu.sync_copy(b2_hbm, b2_v)

    neg = jnp.full((16,), -jnp.inf, jnp.float32)
    for g in range(G):
        for k in range(H // 16):
            pooled_v[g, pl.ds(16 * k, 16)] = neg

    @pl.when(wid < NW - 1)
    def _full():
        pltpu.sync_copy(parts_hbm.at[pl.ds(row0, PRS)], p_v)
        pltpu.sync_copy(u_hbm.at[pl.ds(row0, PRS)], u_v)
        pltpu.sync_copy(batch_hbm.at[pl.ds(row0, PRS)], b_v)
        pltpu.sync_copy(dinv_hbm.at[pl.ds(row0, PRS), pl.ds(0, 16)], d_v)
        _pool_rows(p_v, u_v, b_v, d_v, b2_v, pooled_v, PRS // 16)

    @pl.when(wid == NW - 1)
    def _tail():
        pltpu.sync_copy(parts_hbm.at[pl.ds(row0, PTAIL)],
                        p_v.at[pl.ds(0, PTAIL)])
        pltpu.sync_copy(u_hbm.at[pl.ds(row0, PTAIL)],
                        u_v.at[pl.ds(0, PTAIL)])
        pltpu.sync_copy(batch_hbm.at[pl.ds(row0, PTAIL)],
                        b_v.at[pl.ds(0, PTAIL)])
        pltpu.sync_copy(dinv_hbm.at[pl.ds(row0, PTAIL), pl.ds(0, 16)],
                        d_v.at[pl.ds(0, PTAIL)])
        _pool_rows(p_v, u_v, b_v, d_v, b2_v, pooled_v, PTAIL // 16)

    pltpu.sync_copy(pooled_v, out_hbm.at[wid, pl.ds(0, G), pl.ds(0, H)])


_pool_call = functools.partial(
    pl.kernel,
    mesh=_mesh,
    compiler_params=_sc_params,
    out_type=jax.ShapeDtypeStruct((NW, G, HW), jnp.float32),
    scratch_types=[
        pltpu.VMEM((PRS, HW), jnp.float32),
        pltpu.VMEM((PRS, HW), jnp.float32),
        pltpu.VMEM((PRS,), jnp.int32),
        pltpu.VMEM((PRS, 16), jnp.float32),
        pltpu.VMEM((H,), jnp.float32),
        pltpu.VMEM((G, H), jnp.float32),
    ],
)(_pool_body)


def _lin1_body(x_ref, w1_ref, dp_ref, u1_ref, dinv_ref):
    deg = 1.0 + dp_ref[:N, 0:1] + dp_ref[:N, DEGW:DEGW + 1]
    dinv = lax.rsqrt(deg)
    dinv_ref[...] = jnp.broadcast_to(dinv, (N, HW))
    u1_ref[...] = dinv * jnp.dot(x_ref[...], w1_ref[...],
                                 preferred_element_type=jnp.float32)


_lin1 = pl.pallas_call(
    _lin1_body,
    out_shape=(jax.ShapeDtypeStruct((N, HW), jnp.float32),
               jax.ShapeDtypeStruct((N, HW), jnp.float32)),
)


def _lin2_body(p_ref, u1_ref, dinv_ref, b1_ref, w2_ref, u2_ref):
    sagg = p_ref[:N, :H] + p_ref[:N, H:]
    dinv = dinv_ref[:, 0:1]
    h = jnp.maximum(dinv * (sagg + u1_ref[:, :H]) + b1_ref[...], 0.0)
    u2_ref[...] = dinv * jnp.dot(h, w2_ref[...],
                                 preferred_element_type=jnp.float32)


_lin2 = pl.pallas_call(
    _lin2_body,
    out_shape=jax.ShapeDtypeStruct((N, HW), jnp.float32),
)


def _head_body(p_ref, w3_ref, b3_ref, out_ref):
    pooled = p_ref[0, :, :H]
    for w in range(1, NW):
        pooled = jnp.maximum(pooled, p_ref[w, :, :H])
    out_ref[...] = jnp.dot(pooled, w3_ref[...],
                           preferred_element_type=jnp.float32) + b3_ref[...]


_head = pl.pallas_call(
    _head_body,
    out_shape=jax.ShapeDtypeStruct((G, 1), jnp.float32),
)


def kernel(x, edge_index, batch_index, W1, b1, W2, b2, W3, b3):
    src = edge_index[0]
    dst = edge_index[1]
    srcp = jnp.pad(src, (0, EPAD - E)).reshape(NW, NJ, CHUNK)
    dstp = jnp.pad(dst, (0, EPAD - E),
                   constant_values=N).reshape(NW, NJ, CHUNK)
    ones = jnp.ones((CHUNK, DEGW), jnp.float32)
    zeros_deg = jnp.zeros((RPT, DEGW), jnp.float32)
    zeros_h = jnp.zeros((RPT, H), jnp.float32)
    W1p = jnp.pad(W1, ((0, 0), (0, HW - H)))
    W2p = jnp.pad(W2, ((0, 0), (0, HW - H)))

    deg_parts = _deg_call(dstp, ones, zeros_deg)
    u1, dinv = _lin1(x, W1p, deg_parts)
    s1 = _agg_call(u1, srcp, dstp, zeros_h)
    u2 = _lin2(s1, u1, dinv, b1.reshape(1, H), W2p)
    s2 = _agg_call(u2, srcp, dstp, zeros_h)
    pooled_parts = _pool_call(s2, u2, batch_index, dinv, b2)
    out = _head(pooled_parts, W3, b3.reshape(1, 1))
    return out


# speedup vs baseline: 1.0143x; 1.0105x over previous
import functools

import jax
import jax.numpy as jnp
from jax import lax
from jax.experimental import pallas as pl
from jax.experimental.pallas import tpu as pltpu
from jax.experimental.pallas import tpu_sc as plsc

N = 10000
E = 320000
F = 128
H = 64
G = 64
HW = 128
NC = 2
NS = 16
NW = NC * NS
CHUNK = 64
NJ = 168
RING = 6
HRING = RING // 2
EPAD = NW * NJ * CHUNK
NP = 10112
RPT = NP // NS
DEGW = 16
PRS = 320
PTAIL = N - (NW - 1) * PRS

_mesh = plsc.VectorSubcoreMesh(core_axis_name="c", subcore_axis_name="s")
_sc_params = pltpu.CompilerParams(use_tc_tiling_on_sc=False)


def _deg_body(dst_hbm, ones_hbm, zeros_hbm, parts_hbm, dst_v, ones_v, acc_sh):
    c = lax.axis_index("c")
    s = lax.axis_index("s")
    wid = s * NC + c
    pltpu.sync_copy(dst_hbm.at[wid], dst_v)
    pltpu.sync_copy(ones_hbm, ones_v)
    row0 = s * RPT
    pltpu.sync_copy(zeros_hbm, acc_sh.at[pl.ds(row0, RPT)])
    plsc.subcore_barrier()

    def body(j, carry):
        pltpu.sync_copy(ones_v, acc_sh.at[dst_v.at[j]], add=True)
        return carry

    lax.fori_loop(0, NJ, body, 0)
    plsc.subcore_barrier()
    pltpu.sync_copy(acc_sh.at[pl.ds(row0, RPT)],
                    parts_hbm.at[pl.ds(row0, RPT), pl.ds(DEGW * c, DEGW)])


_deg_call = functools.partial(
    pl.kernel,
    mesh=_mesh,
    compiler_params=_sc_params,
    out_type=jax.ShapeDtypeStruct((NP, HW), jnp.float32),
    scratch_types=[
        pltpu.VMEM((NJ, CHUNK), jnp.int32),
        pltpu.VMEM((CHUNK, DEGW), jnp.float32),
        pltpu.VMEM_SHARED((NP, DEGW), jnp.float32),
    ],
)(_deg_body)


def _agg_body(u_hbm, src_hbm, dst_hbm, zeros_hbm, parts_hbm,
              src_v, dst_v, *rest):
    bufs = rest[:RING]
    gsem = rest[RING:2 * RING]
    ssem = rest[2 * RING:3 * RING]
    u_sh = rest[3 * RING]
    acc_sh = rest[3 * RING + 1]
    c = lax.axis_index("c")
    s = lax.axis_index("s")
    wid = s * NC + c
    pltpu.sync_copy(src_hbm.at[wid], src_v)
    pltpu.sync_copy(dst_hbm.at[wid], dst_v)
    urow0 = s * (N // NS)
    pltpu.sync_copy(u_hbm.at[pl.ds(urow0, N // NS), pl.ds(0, H)],
                    u_sh.at[pl.ds(urow0, N // NS)])
    row0 = s * RPT
    pltpu.sync_copy(zeros_hbm, acc_sh.at[pl.ds(row0, RPT)])
    plsc.subcore_barrier()

    def start_gather(j, k):
        pltpu.async_copy(u_sh.at[src_v.at[j]], bufs[k], gsem[k])

    def wait_gather(j, k):
        pltpu.make_async_copy(u_sh.at[src_v.at[j]], bufs[k], gsem[k]).wait()

    def start_scatter(j, k):
        pltpu.async_copy(bufs[k], acc_sh.at[dst_v.at[j]], ssem[k], add=True)

    def wait_scatter(j, k):
        pltpu.make_async_copy(bufs[k], acc_sh.at[dst_v.at[j]],
                              ssem[k]).wait()

    for k in range(HRING):
        start_gather(k, k)
    for j in range(HRING):
        wait_gather(j, j)
        start_scatter(j, j)
        start_gather(j + HRING, j + HRING)

    def body(t, carry):
        for k in range(RING):
            j = HRING + t * RING + k
            kj = (HRING + k) % RING
            wait_gather(j, kj)
            start_scatter(j, kj)
            wait_scatter(j - HRING, k)
            start_gather(j + HRING, k)
        return carry

    lax.fori_loop(0, (NJ - RING) // RING, body, 0)

    for j in range(NJ - HRING, NJ):
        wait_gather(j, j % RING)
        start_scatter(j, j % RING)
    for j in range(NJ - RING, NJ):
        wait_scatter(j, j % RING)

    plsc.subcore_barrier()
    pltpu.sync_copy(acc_sh.at[pl.ds(row0, RPT)],
                    parts_hbm.at[pl.ds(row0, RPT), pl.ds(H * c, H)])


_agg_call = functools.partial(
    pl.kernel,
    mesh=_mesh,
    compiler_params=_sc_params,
    out_type=jax.ShapeDtypeStruct((NP, HW), jnp.float32),
    scratch_types=(
        [pltpu.VMEM((NJ, CHUNK), jnp.int32),
         pltpu.VMEM((NJ, CHUNK), jnp.int32)]
        + [pltpu.VMEM((CHUNK, H), jnp.float32) for _ in range(RING)]
        + [pltpu.SemaphoreType.DMA for _ in range(2 * RING)]
        + [pltpu.VMEM_SHARED((N, H), jnp.float32),
           pltpu.VMEM_SHARED((NP, H), jnp.float32)]
    ),
)(_agg_body)


def _pool_rows(p_v, u_v, b_v, d_v, b2_v, pooled_v, ngroups):
    def grp(t, carry):
        base = 16 * t
        bvec = b_v[pl.ds(base, 16)]
        for lane in range(16):
            i = base + lane
            g = bvec[lane]
            dinv = d_v[i, pl.ds(0, 16)][0]
            for k in range(H // 16):
                cl = 16 * k
                v = (p_v[i, pl.ds(cl, 16)] + p_v[i, pl.ds(H + cl, 16)]
                     + u_v[i, pl.ds(cl, 16)])
                h2 = jnp.maximum(v * dinv + b2_v[pl.ds(cl, 16)], 0.0)
                pooled_v[g, pl.ds(cl, 16)] = jnp.maximum(
                    pooled_v[g, pl.ds(cl, 16)], h2)
        return carry

    lax.fori_loop(0, ngroups, grp, 0)


def _pool_body(parts_hbm, u_hbm, batch_hbm, dinv_hbm, b2_hbm, out_hbm,
               p_v, u_v, b_v, d_v, b2_v, pooled_v):
    c = lax.axis_index("c")
    s = lax.axis_index("s")
    wid = s * NC + c
    row0 = wid * PRS
    pltpu.sync_copy(b2_hbm, b2_v)

    neg = jnp.full((16,), -jnp.inf, jnp.float32)
    for g in range(G):
        for k in range(H // 16):
            pooled_v[g, pl.ds(16 * k, 16)] = neg

    @pl.when(wid < NW - 1)
    def _full():
        pltpu.sync_copy(parts_hbm.at[pl.ds(row0, PRS)], p_v)
        pltpu.sync_copy(u_hbm.at[pl.ds(row0, PRS)], u_v)
        pltpu.sync_copy(batch_hbm.at[pl.ds(row0, PRS)], b_v)
        pltpu.sync_copy(dinv_hbm.at[pl.ds(row0, PRS), pl.ds(0, 16)], d_v)
        _pool_rows(p_v, u_v, b_v, d_v, b2_v, pooled_v, PRS // 16)

    @pl.when(wid == NW - 1)
    def _tail():
        pltpu.sync_copy(parts_hbm.at[pl.ds(row0, PTAIL)],
                        p_v.at[pl.ds(0, PTAIL)])
        pltpu.sync_copy(u_hbm.at[pl.ds(row0, PTAIL)],
                        u_v.at[pl.ds(0, PTAIL)])
        pltpu.sync_copy(batch_hbm.at[pl.ds(row0, PTAIL)],
                        b_v.at[pl.ds(0, PTAIL)])
        pltpu.sync_copy(dinv_hbm.at[pl.ds(row0, PTAIL), pl.ds(0, 16)],
                        d_v.at[pl.ds(0, PTAIL)])
        _pool_rows(p_v, u_v, b_v, d_v, b2_v, pooled_v, PTAIL // 16)

    pltpu.sync_copy(pooled_v, out_hbm.at[wid, pl.ds(0, G), pl.ds(0, H)])


_pool_call = functools.partial(
    pl.kernel,
    mesh=_mesh,
    compiler_params=_sc_params,
    out_type=jax.ShapeDtypeStruct((NW, G, HW), jnp.float32),
    scratch_types=[
        pltpu.VMEM((PRS, HW), jnp.float32),
        pltpu.VMEM((PRS, HW), jnp.float32),
        pltpu.VMEM((PRS,), jnp.int32),
        pltpu.VMEM((PRS, 16), jnp.float32),
        pltpu.VMEM((H,), jnp.float32),
        pltpu.VMEM((G, H), jnp.float32),
    ],
)(_pool_body)


def _lin1_body(x_ref, w1_ref, dp_ref, u1_ref, dinv_ref):
    deg = 1.0 + dp_ref[:N, 0:1] + dp_ref[:N, DEGW:DEGW + 1]
    dinv = lax.rsqrt(deg)
    dinv_ref[...] = jnp.broadcast_to(dinv, (N, HW))
    u1_ref[...] = dinv * jnp.dot(x_ref[...], w1_ref[...],
                                 preferred_element_type=jnp.float32)


_lin1 = pl.pallas_call(
    _lin1_body,
    out_shape=(jax.ShapeDtypeStruct((N, HW), jnp.float32),
               jax.ShapeDtypeStruct((N, HW), jnp.float32)),
)


def _lin2_body(p_ref, u1_ref, dinv_ref, b1_ref, w2_ref, u2_ref):
    sagg = p_ref[:N, :H] + p_ref[:N, H:]
    dinv = dinv_ref[:, 0:1]
    h = jnp.maximum(dinv * (sagg + u1_ref[:, :H]) + b1_ref[...], 0.0)
    u2_ref[...] = dinv * jnp.dot(h, w2_ref[...],
                                 preferred_element_type=jnp.float32)


_lin2 = pl.pallas_call(
    _lin2_body,
    out_shape=jax.ShapeDtypeStruct((N, HW), jnp.float32),
)


def _head_body(p_ref, w3_ref, b3_ref, out_ref):
    pooled = p_ref[0, :, :H]
    for w in range(1, NW):
        pooled = jnp.maximum(pooled, p_ref[w, :, :H])
    out_ref[...] = jnp.dot(pooled, w3_ref[...],
                           preferred_element_type=jnp.float32) + b3_ref[...]


_head = pl.pallas_call(
    _head_body,
    out_shape=jax.ShapeDtypeStruct((G, 1), jnp.float32),
)


def kernel(x, edge_index, batch_index, W1, b1, W2, b2, W3, b3):
    src = edge_index[0]
    dst = edge_index[1]
    srcp = jnp.pad(src, (0, EPAD - E)).reshape(NW, NJ, CHUNK)
    dstp = jnp.pad(dst, (0, EPAD - E),
                   constant_values=N).reshape(NW, NJ, CHUNK)
    ones = jnp.ones((CHUNK, DEGW), jnp.float32)
    zeros_deg = jnp.zeros((RPT, DEGW), jnp.float32)
    zeros_h = jnp.zeros((RPT, H), jnp.float32)
    W1p = jnp.pad(W1, ((0, 0), (0, HW - H)))
    W2p = jnp.pad(W2, ((0, 0), (0, HW - H)))

    deg_parts = _deg_call(dstp, ones, zeros_deg)
    u1, dinv = _lin1(x, W1p, deg_parts)
    s1 = _agg_call(u1, srcp, dstp, zeros_h)
    u2 = _lin2(s1, u1, dinv, b1.reshape(1, H), W2p)
    s2 = _agg_call(u2, srcp, dstp, zeros_h)
    pooled_parts = _pool_call(s2, u2, batch_index, dinv, b2)
    out = _head(pooled_parts, W3, b3.reshape(1, 1))
    return out
